# baseline (speedup 1.0000x reference)
import functools

import jax
import jax.numpy as jnp
from jax import lax
from jax.experimental import pallas as pl
from jax.experimental.pallas import tpu as pltpu
from jax.experimental.pallas import tpu_sc as plsc

B, S, D = 16, 512, 768
NL = 4
NSTRIP = 16
SW = D // NSTRIP
NROWS = NL * B * S * NSTRIP
REG = 2 * S
K = 32
IDXLEN = B * REG + 2 * K
LANES = 16
NSEG = 2 * B

_mesh = plsc.VectorSubcoreMesh(core_axis_name="c", subcore_axis_name="s")


def _al8(x):
    return pl.multiple_of(x, 8)


def _lane_iota():
    return jnp.arange(LANES, dtype=jnp.int32)


def _p1_body(table, ids_h, am_h, typ_h, part_h, cnt_h,
             row_i, row_a, row_t, bld, msk, idxb, gb, parts, cntv, cstage,
             seg_st, seg_ln, sh_idx, sh_cnt, sem0):
    ci = lax.axis_index("c")
    si = lax.axis_index("s")
    b = si
    iota = _lane_iota()
    zero16 = jnp.zeros((LANES,), jnp.int32)
    zf = jnp.zeros((LANES,), jnp.float32)

    pltpu.sync_copy(ids_h.at[b], row_i)
    pltpu.sync_copy(am_h.at[b], row_a)
    pltpu.sync_copy(typ_h.at[b], row_t)

    def _zb(i, _):
        bld[pl.ds(_al8(i * LANES), LANES)] = zero16
        return 0
    lax.fori_loop(0, (REG + LANES) // LANES, _zb, 0)

    c_l0 = ((2 * ci) * B + b) * S * NSTRIP
    c_l1 = ((2 * ci + 1) * B + b) * S * NSTRIP

    def _pass1(v, carry):
        zcarry, off, nt = carry
        o = _al8(v * LANES)
        iv = row_i[pl.ds(o, LANES)]
        av = row_a[pl.ds(o, LANES)]
        tv = row_t[pl.ds(o, LANES)]
        z = (iv == 0).astype(jnp.int32)
        cs = jnp.cumsum(z) + zcarry
        valid = (cs == 0) & (iv != 101) & (iv != 102) & (av == 1)
        mterm = valid & (tv == 0)
        mtext = valid & (tv == 1)
        msk[pl.ds(o, LANES)] = mtext.astype(jnp.int32)
        pos16 = (iota + v * LANES) * NSTRIP
        mi = mterm.astype(jnp.int32)
        h = jnp.sum(mi)
        pr = jnp.cumsum(mi) - mi + off
        plsc.store_scatter(bld, [pr], pos16 + c_l0, mask=mterm)
        plsc.store_scatter(bld, [pr + h], pos16 + c_l1, mask=mterm)
        return zcarry + jnp.sum(z), off + 2 * h, nt + h

    _, off, nt = lax.fori_loop(0, S // LANES, _pass1, (0, 0, 0))
    off = (off + 7) & (-8)

    def _pass2(v, carry):
        off, nx = carry
        o = _al8(v * LANES)
        mtext = msk[pl.ds(o, LANES)] != 0
        pos16 = (iota + v * LANES) * NSTRIP
        mi = mtext.astype(jnp.int32)
        h = jnp.sum(mi)
        pr = jnp.cumsum(mi) - mi + off
        plsc.store_scatter(bld, [pr], pos16 + c_l0, mask=mtext)
        plsc.store_scatter(bld, [pr + h], pos16 + c_l1, mask=mtext)
        return off + 2 * h, nx + h

    _, nx = lax.fori_loop(0, S // LANES, _pass2, (off, 0))

    cstage[...] = (jnp.where(iota == 0, nt, 0)
                   + jnp.where(iota == 1, nx, 0))
    pltpu.sync_copy(cstage, sh_cnt.at[b])
    pltpu.sync_copy(bld.at[pl.ds(0, REG)], sh_idx.at[b])

    @pl.when(ci == 0)
    def _():
        pltpu.sync_copy(cstage, cnt_h.at[b])

    plsc.subcore_barrier()

    pltpu.sync_copy(sh_cnt, cntv)
    for bb in range(B):
        pltpu.sync_copy(sh_idx.at[bb], idxb.at[pl.ds(bb * REG, REG)])
    for i in range(2 * K // LANES):
        idxb[pl.ds(B * REG + i * LANES, LANES)] = zero16

    @pl.when(si > 0)
    def _():
        def _adds(i, _):
            o = _al8(i * LANES)
            idxb[pl.ds(o, LANES)] = idxb[pl.ds(o, LANES)] + si
            return 0
        lax.fori_loop(0, B * REG // LANES, _adds, 0)

    for bb in range(B):
        row = cntv[bb]
        ntb = jnp.sum(jnp.where(iota == 0, row, 0))
        nxb = jnp.sum(jnp.where(iota == 1, row, 0))
        t1s = (2 * ntb + 7) & (-8)
        seg_st[2 * bb] = bb * REG
        seg_ln[2 * bb] = 2 * ntb
        seg_st[2 * bb + 1] = bb * REG + t1s
        seg_ln[2 * bb + 1] = 2 * nxb

    def _seg(g, _):
        st = seg_st[g]
        ln = seg_ln[g]
        nch = (ln + K - 1) // K

        def _chunk(i, accs):
            pltpu.async_copy(table.at[idxb.at[pl.ds(_al8(st + i * K), K)]],
                             gb, sem0).wait()
            base = i * K
            accs = list(accs)
            for j in range(K):
                bank = 3 * (j % 2)
                c = base + j < ln
                for k3 in range(3):
                    r = gb[j, pl.ds(k3 * LANES, LANES)]
                    accs[bank + k3] = accs[bank + k3] + jnp.where(c, r, zf)
            return tuple(accs)

        accs = lax.fori_loop(0, nch, _chunk, (zf, zf, zf, zf, zf, zf))
        for k3 in range(3):
            parts[pl.ds(_al8(g * SW + k3 * LANES), LANES)] = accs[k3] + accs[3 + k3]
        return 0

    lax.fori_loop(0, NSEG, _seg, 0)
    woff = _al8((ci * NSTRIP + si) * (NSEG * SW))
    pltpu.sync_copy(parts, part_h.at[pl.ds(woff, NSEG * SW)])


def _p2_body(part_h, cnt_h, out_h, sab, accv, crow, sem0):
    ci = lax.axis_index("c")
    si = lax.axis_index("s")
    b = si
    t = ci
    g = 2 * b + t
    iota = _lane_iota()

    pltpu.sync_copy(cnt_h.at[b], crow)
    cv = crow[...]
    n = jnp.where(t == 0, cv[0], cv[1])
    nv = jnp.broadcast_to(n.astype(jnp.float32), (LANES,))
    scale = jnp.float32(0.25) / nv

    cps = []
    for w in range(2 * NSTRIP):
        off = _al8(w * (NSEG * SW) + g * SW)
        cps.append(pltpu.async_copy(part_h.at[pl.ds(off, SW)], sab.at[w], sem0))
    for cp in cps:
        cp.wait()
    for s2 in range(NSTRIP):
        for k3 in range(3):
            v = (sab[s2, pl.ds(k3 * LANES, LANES)]
                 + sab[NSTRIP + s2, pl.ds(k3 * LANES, LANES)])
            accv[pl.ds(s2 * SW + k3 * LANES, LANES)] = v * scale
    pltpu.sync_copy(accv, out_h.at[b, pl.ds(_al8(t * D), D)])


_phase1 = functools.partial(
    pl.kernel,
    out_type=[jax.ShapeDtypeStruct((2 * NSTRIP * NSEG * SW,), jnp.float32),
              jax.ShapeDtypeStruct((B, LANES), jnp.int32)],
    mesh=_mesh,
    compiler_params=pltpu.CompilerParams(needs_layout_passes=False,
                                         use_tc_tiling_on_sc=False),
    scratch_types=[
        pltpu.VMEM((S,), jnp.int32),
        pltpu.VMEM((S,), jnp.int32),
        pltpu.VMEM((S,), jnp.int32),
        pltpu.VMEM((REG + LANES,), jnp.int32),
        pltpu.VMEM((S,), jnp.int32),
        pltpu.VMEM((IDXLEN,), jnp.int32),
        pltpu.VMEM((K, SW), jnp.float32),
        pltpu.VMEM((NSEG * SW,), jnp.float32),
        pltpu.VMEM((B, LANES), jnp.int32),
        pltpu.VMEM((LANES,), jnp.int32),
        pltpu.SMEM((NSEG,), jnp.int32),
        pltpu.SMEM((NSEG,), jnp.int32),
        pltpu.VMEM_SHARED((B, REG), jnp.int32),
        pltpu.VMEM_SHARED((B, LANES), jnp.int32),
        pltpu.SemaphoreType.DMA,
    ],
)(_p1_body)

_phase2 = functools.partial(
    pl.kernel,
    out_type=jax.ShapeDtypeStruct((B, 2 * D), jnp.float32),
    mesh=_mesh,
    compiler_params=pltpu.CompilerParams(needs_layout_passes=False),
    scratch_types=[
        pltpu.VMEM((2 * NSTRIP, SW), jnp.float32),
        pltpu.VMEM((D,), jnp.float32),
        pltpu.VMEM((LANES,), jnp.int32),
        pltpu.SemaphoreType.DMA,
    ],
)(_p2_body)


@jax.jit
def kernel(bert_out, input_ids, attention_mask, token_type_ids):
    table = bert_out[-NL:].reshape(NROWS, SW)
    ids32 = input_ids.astype(jnp.int32)
    am32 = attention_mask.astype(jnp.int32)
    typ32 = token_type_ids.astype(jnp.int32)
    part, cnt = _phase1(table, ids32, am32, typ32)
    return _phase2(part, cnt)


# speedup vs baseline: 1.3318x; 1.3318x over previous
import functools

import jax
import jax.numpy as jnp
from jax import lax
from jax.experimental import pallas as pl
from jax.experimental.pallas import tpu as pltpu
from jax.experimental.pallas import tpu_sc as plsc

B, S, D = 16, 512, 768
NL = 4
NSTRIP = 16
SW = D // NSTRIP
NROWS = NL * B * S * NSTRIP
REG = 2 * S
K = 32
IDXLEN = B * REG + 2 * K
LANES = 16
NSEG = 2 * B
NCHMAX = B * REG // K + 2 * NSEG

_mesh = plsc.VectorSubcoreMesh(core_axis_name="c", subcore_axis_name="s")


def _al8(x):
    return pl.multiple_of(x, 8)


def _lane_iota():
    return jnp.arange(LANES, dtype=jnp.int32)


def _p1_body(table, ids_h, am_h, typ_h, part_h, cnt_h,
             row_i, row_a, row_t, bld, msk, idxb, gb, gb2, parts, cntv, cstage,
             wl, sh_idx, sh_cnt, sem0, sem1):
    ci = lax.axis_index("c")
    si = lax.axis_index("s")
    b = si
    iota = _lane_iota()
    zero16 = jnp.zeros((LANES,), jnp.int32)
    zf = jnp.zeros((LANES,), jnp.float32)

    pltpu.sync_copy(ids_h.at[b], row_i)
    pltpu.sync_copy(am_h.at[b], row_a)
    pltpu.sync_copy(typ_h.at[b], row_t)

    def _zb(i, _):
        bld[pl.ds(_al8(i * LANES), LANES)] = zero16
        return 0
    lax.fori_loop(0, (REG + LANES) // LANES, _zb, 0)

    c_l0 = ((2 * ci) * B + b) * S * NSTRIP
    c_l1 = ((2 * ci + 1) * B + b) * S * NSTRIP

    def _pass1(v, carry):
        zcarry, off, nt = carry
        o = _al8(v * LANES)
        iv = row_i[pl.ds(o, LANES)]
        av = row_a[pl.ds(o, LANES)]
        tv = row_t[pl.ds(o, LANES)]
        z = (iv == 0).astype(jnp.int32)
        cs = jnp.cumsum(z) + zcarry
        valid = (cs == 0) & (iv != 101) & (iv != 102) & (av == 1)
        mterm = valid & (tv == 0)
        mtext = valid & (tv == 1)
        msk[pl.ds(o, LANES)] = mtext.astype(jnp.int32)
        pos16 = (iota + v * LANES) * NSTRIP
        mi = mterm.astype(jnp.int32)
        h = jnp.sum(mi)
        pr = jnp.cumsum(mi) - mi + off
        plsc.store_scatter(bld, [pr], pos16 + c_l0, mask=mterm)
        plsc.store_scatter(bld, [pr + h], pos16 + c_l1, mask=mterm)
        return zcarry + jnp.sum(z), off + 2 * h, nt + h

    _, off, nt = lax.fori_loop(0, S // LANES, _pass1, (0, 0, 0))
    off = (off + K - 1) & (-K)

    def _pass2(v, carry):
        off, nx = carry
        o = _al8(v * LANES)
        mtext = msk[pl.ds(o, LANES)] != 0
        pos16 = (iota + v * LANES) * NSTRIP
        mi = mtext.astype(jnp.int32)
        h = jnp.sum(mi)
        pr = jnp.cumsum(mi) - mi + off
        plsc.store_scatter(bld, [pr], pos16 + c_l0, mask=mtext)
        plsc.store_scatter(bld, [pr + h], pos16 + c_l1, mask=mtext)
        return off + 2 * h, nx + h

    _, nx = lax.fori_loop(0, S // LANES, _pass2, (off, 0))

    cstage[...] = (jnp.where(iota == 0, nt, 0)
                   + jnp.where(iota == 1, nx, 0))
    pltpu.sync_copy(cstage, sh_cnt.at[b])
    pltpu.sync_copy(bld.at[pl.ds(0, REG)], sh_idx.at[b])

    @pl.when(ci == 0)
    def _():
        pltpu.sync_copy(cstage, cnt_h.at[b])

    plsc.subcore_barrier()

    pltpu.sync_copy(sh_cnt, cntv)
    for bb in range(B):
        pltpu.sync_copy(sh_idx.at[bb], idxb.at[pl.ds(bb * REG, REG)])
    for i in range(2 * K // LANES):
        idxb[pl.ds(B * REG + i * LANES, LANES)] = zero16

    @pl.when(si > 0)
    def _():
        def _adds(i, _):
            o = _al8(i * LANES)
            idxb[pl.ds(o, LANES)] = idxb[pl.ds(o, LANES)] + si
            return 0
        lax.fori_loop(0, B * REG // LANES, _adds, 0)

    def _zp(i, _):
        parts[pl.ds(_al8(i * LANES), LANES)] = zf
        return 0
    lax.fori_loop(0, NSEG * SW // LANES, _zp, 0)

    cc = 0
    for bb in range(B):
        row = cntv[bb]
        ntb = row[0]
        nxb = row[1]
        t1s = (2 * ntb + K - 1) & (-K)
        for t in range(2):
            stt = bb * REG + t1s * t
            ln = jnp.where(t == 0, 2 * ntb, 2 * nxb)
            seg = 2 * bb + t
            nch = (ln + K - 1) // K

            def _wl(j, cc, stt=stt, ln=ln, seg=seg):
                word = (stt + j * K) | (seg << 15) | (jnp.minimum(ln - j * K, 63) << 20)
                wl[cc] = word
                return cc + 1
            cc = lax.fori_loop(0, nch, _wl, cc)
    wl[cc] = 0
    wl[cc + 1] = 0

    def _issue(word, buf, sem):
        o = pl.multiple_of(word & 32767, 8)
        pltpu.async_copy(table.at[idxb.at[pl.ds(o, K)]], buf, sem)

    def _wait(word, buf, sem):
        o = pl.multiple_of(word & 32767, 8)
        pltpu.make_async_copy(table.at[idxb.at[pl.ds(o, K)]], buf, sem).wait()

    def _fold_into_parts(a, seg):
        for k3 in range(3):
            plsc.addupdate(parts.at[pl.ds(_al8(seg * SW + k3 * LANES), LANES)],
                           a[k3] + a[3 + k3])

    def _acc_full(buf, seg):
        a = [zf] * 6
        for j in range(K):
            bank = 3 * (j % 2)
            for k3 in range(3):
                a[bank + k3] = a[bank + k3] + buf[j, pl.ds(k3 * LANES, LANES)]
        _fold_into_parts(a, seg)

    def _acc_masked(buf, seg, lnr):
        a = [zf] * 6
        for j in range(K):
            bank = 3 * (j % 2)
            c = j < lnr
            for k3 in range(3):
                r = buf[j, pl.ds(k3 * LANES, LANES)]
                a[bank + k3] = a[bank + k3] + jnp.where(c, r, zf)
        _fold_into_parts(a, seg)

    def _acc_buf(buf, word):
        lnr = (word >> 20) & 63
        seg = (word >> 15) & 31

        @pl.when(lnr >= K)
        def _():
            _acc_full(buf, seg)

        @pl.when(lnr < K)
        def _():
            _acc_masked(buf, seg, lnr)

    nprs = (cc + 1) // 2
    _issue(wl[0], gb, sem0)

    def _pair(i2, _):
        wb = wl[2 * i2 + 1]
        _issue(wb, gb2, sem1)
        wa = wl[2 * i2]
        _wait(wa, gb, sem0)
        _acc_buf(gb, wa)
        _issue(wl[2 * i2 + 2], gb, sem0)
        _wait(wb, gb2, sem1)
        _acc_buf(gb2, wb)
        return 0

    lax.fori_loop(0, nprs, _pair, 0)
    _wait(wl[2 * nprs], gb, sem0)

    woff = _al8((ci * NSTRIP + si) * (NSEG * SW))
    pltpu.sync_copy(parts, part_h.at[pl.ds(woff, NSEG * SW)])


def _p2_body(part_h, cnt_h, out_h, sab, accv, crow, sem0):
    ci = lax.axis_index("c")
    si = lax.axis_index("s")
    b = si
    t = ci
    g = 2 * b + t
    iota = _lane_iota()

    pltpu.sync_copy(cnt_h.at[b], crow)
    cv = crow[...]
    n = jnp.where(t == 0, cv[0], cv[1])
    nv = jnp.broadcast_to(n.astype(jnp.float32), (LANES,))
    scale = jnp.float32(0.25) / nv

    cps = []
    for w in range(2 * NSTRIP):
        off = _al8(w * (NSEG * SW) + g * SW)
        cps.append(pltpu.async_copy(part_h.at[pl.ds(off, SW)], sab.at[w], sem0))
    for cp in cps:
        cp.wait()
    for s2 in range(NSTRIP):
        for k3 in range(3):
            v = (sab[s2, pl.ds(k3 * LANES, LANES)]
                 + sab[NSTRIP + s2, pl.ds(k3 * LANES, LANES)])
            accv[pl.ds(s2 * SW + k3 * LANES, LANES)] = v * scale
    pltpu.sync_copy(accv, out_h.at[b, pl.ds(_al8(t * D), D)])


_phase1 = functools.partial(
    pl.kernel,
    out_type=[jax.ShapeDtypeStruct((2 * NSTRIP * NSEG * SW,), jnp.float32),
              jax.ShapeDtypeStruct((B, LANES), jnp.int32)],
    mesh=_mesh,
    compiler_params=pltpu.CompilerParams(needs_layout_passes=False,
                                         use_tc_tiling_on_sc=False),
    scratch_types=[
        pltpu.VMEM((S,), jnp.int32),
        pltpu.VMEM((S,), jnp.int32),
        pltpu.VMEM((S,), jnp.int32),
        pltpu.VMEM((REG + LANES,), jnp.int32),
        pltpu.VMEM((S,), jnp.int32),
        pltpu.VMEM((IDXLEN,), jnp.int32),
        pltpu.VMEM((K, SW), jnp.float32),
        pltpu.VMEM((K, SW), jnp.float32),
        pltpu.VMEM((NSEG * SW,), jnp.float32),
        pltpu.VMEM((B, LANES), jnp.int32),
        pltpu.VMEM((LANES,), jnp.int32),
        pltpu.SMEM((NCHMAX + 2,), jnp.int32),
        pltpu.VMEM_SHARED((B, REG), jnp.int32),
        pltpu.VMEM_SHARED((B, LANES), jnp.int32),
        pltpu.SemaphoreType.DMA,
        pltpu.SemaphoreType.DMA,
    ],
)(_p1_body)

_phase2 = functools.partial(
    pl.kernel,
    out_type=jax.ShapeDtypeStruct((B, 2 * D), jnp.float32),
    mesh=_mesh,
    compiler_params=pltpu.CompilerParams(needs_layout_passes=False),
    scratch_types=[
        pltpu.VMEM((2 * NSTRIP, SW), jnp.float32),
        pltpu.VMEM((D,), jnp.float32),
        pltpu.VMEM((LANES,), jnp.int32),
        pltpu.SemaphoreType.DMA,
    ],
)(_p2_body)


@jax.jit
def kernel(bert_out, input_ids, attention_mask, token_type_ids):
    table = bert_out[-NL:].reshape(NROWS, SW)
    ids32 = input_ids.astype(jnp.int32)
    am32 = attention_mask.astype(jnp.int32)
    typ32 = token_type_ids.astype(jnp.int32)
    part, cnt = _phase1(table, ids32, am32, typ32)
    return _phase2(part, cnt)
